# baseline (speedup 1.0000x reference)
import functools

import jax
import jax.numpy as jnp
from jax import lax
from jax.experimental import pallas as pl
from jax.experimental.pallas import tpu as pltpu
from jax.experimental.pallas import tpu_sc as plsc


def _dequant_body(w_ref, s_ref, o_ref):
    w = w_ref[...].astype(jnp.float32)
    s = s_ref[...]
    r = lax.broadcasted_iota(jnp.int32, (8, 128), 0)
    l = lax.broadcasted_iota(jnp.int32, (8, 128), 1)
    m = (l // 16 == r).astype(jnp.float32)
    s_exp = lax.dot_general(s, m, (((1,), (0,)), ((), ())),
                            preferred_element_type=jnp.float32)
    o_ref[...] = w * s_exp


def _dequantize(w8, s8, blk):
    rows = w8.shape[0]
    grid = rows // blk
    return pl.pallas_call(
        _dequant_body,
        grid=(grid,),
        in_specs=[
            pl.BlockSpec((blk, 128), lambda i: (i, 0)),
            pl.BlockSpec((blk, 8), lambda i: (i, 0)),
        ],
        out_specs=pl.BlockSpec((blk, 128), lambda i: (i, 0)),
        out_shape=jax.ShapeDtypeStruct((rows, 128), jnp.float32),
    )(w8, s8)


def _sc_gather(table, idx_flat, d):
    info = plsc.get_sparse_core_info()
    nc, ns = info.num_cores, info.num_subcores
    nw = nc * ns
    n = idx_flat.shape[0]
    per_w = n // nw
    ch = 1600
    n_ch = per_w // ch
    mesh = plsc.VectorSubcoreMesh(core_axis_name="c", subcore_axis_name="s")

    @functools.partial(
        pl.kernel,
        mesh=mesh,
        out_type=jax.ShapeDtypeStruct((n, d), jnp.float32),
        scratch_types=[
            pltpu.VMEM((ch,), jnp.int32),
            pltpu.VMEM((ch, d), jnp.float32),
            pltpu.SemaphoreType.DMA,
        ],
        compiler_params=pltpu.CompilerParams(use_tc_tiling_on_sc=False),
    )
    def k(table_hbm, idx_hbm, out_hbm, idx_v, rows_v, sem):
        wid = lax.axis_index("s") * nc + lax.axis_index("c")
        base = wid * per_w

        def body(i, carry):
            off = base + i * ch
            pltpu.sync_copy(idx_hbm.at[pl.ds(off, ch)], idx_v)
            pltpu.async_copy(table_hbm.at[idx_v], rows_v, sem).wait()
            pltpu.sync_copy(rows_v, out_hbm.at[pl.ds(off, ch)])
            return carry

        lax.fori_loop(0, n_ch, body, 0)

    return k(table, idx_flat)


def kernel(input, weight, weight_scale):
    v, d = weight.shape
    w8 = weight.reshape(v // 8, 8 * d)
    s8 = weight_scale.reshape(v // 8, 8)
    table = _dequantize(w8, s8, blk=1000).reshape(v, d)
    idx = input.reshape(-1)
    out = _sc_gather(table, idx, d)
    return out.reshape(*input.shape, d)
